# Initial kernel scaffold; baseline (speedup 1.0000x reference)
import jax
import jax.numpy as jnp
from jax.experimental import pallas as pl


def kernel(x, rel_embeds):
    raise NotImplementedError("write your pallas kernel here")


# speedup vs baseline: 8.1190x; 8.1190x over previous
import functools

import jax
import jax.numpy as jnp
from jax import lax
from jax.experimental import pallas as pl
from jax.experimental.pallas import tpu as pltpu
from jax.experimental.pallas import tpu_sc as plsc

MAXREL = 128
D = 32
T = 2 * MAXREL + 1
NC = 2
NS = 16
NW = NC * NS


def _make_sc_kernel(L: int):
    rows_per_w = L // NW
    win = L + rows_per_w - 1
    win_pad = (win + 7) // 8 * 8

    mesh = plsc.VectorSubcoreMesh(
        core_axis_name="c", subcore_axis_name="s",
        num_cores=NC, num_subcores=NS)

    @functools.partial(
        pl.kernel,
        out_type=jax.ShapeDtypeStruct((L, L, D), jnp.float32),
        mesh=mesh,
        scratch_types=[
            pltpu.VMEM((T, D), jnp.float32),
            pltpu.VMEM((win_pad, D), jnp.float32),
            pltpu.SemaphoreType.DMA,
        ],
        compiler_params=pltpu.CompilerParams(use_tc_tiling_on_sc=False),
    )
    def retrieve(table_hbm, out_hbm, table_v, win_v, sem):
        c = lax.axis_index("c")
        s = lax.axis_index("s")
        wid = c * NS + s
        i0 = wid * rows_per_w

        pltpu.sync_copy(table_hbm, table_v)

        off = MAXREL - i0 - (rows_per_w - 1)

        def build_row(w, carry):
            src = jnp.clip(w + off, 0, T - 1)
            win_v[w, pl.ds(0, 16)] = table_v[src, pl.ds(0, 16)]
            win_v[w, pl.ds(16, 16)] = table_v[src, pl.ds(16, 16)]
            return carry

        lax.fori_loop(0, win, build_row, 0)

        copies = []
        for r in range(rows_per_w):
            copies.append(
                pltpu.async_copy(
                    win_v.at[pl.ds(rows_per_w - 1 - r, L)],
                    out_hbm.at[i0 + r],
                    sem,
                ))
        for cp in copies:
            cp.wait()

    return retrieve


def kernel(x, rel_embeds):
    L = x.shape[1]
    return _make_sc_kernel(L)(rel_embeds)


# speedup vs baseline: 8.1221x; 1.0004x over previous
import functools

import jax
import jax.numpy as jnp
from jax import lax
from jax.experimental import pallas as pl
from jax.experimental.pallas import tpu as pltpu
from jax.experimental.pallas import tpu_sc as plsc

MAXREL = 128
D = 32
T = 2 * MAXREL + 1
NC = 2
NS = 16
NW = NC * NS


def _make_sc_kernel(L: int):
    rows_per_w = L // NW
    win = L + rows_per_w - 1
    win_pad = (win + 7) // 8 * 8

    mesh = plsc.VectorSubcoreMesh(
        core_axis_name="c", subcore_axis_name="s",
        num_cores=NC, num_subcores=NS)

    @functools.partial(
        pl.kernel,
        out_type=jax.ShapeDtypeStruct((L * L * D,), jnp.float32),
        mesh=mesh,
        scratch_types=[
            pltpu.VMEM((T * D,), jnp.float32),
            pltpu.VMEM((win_pad * D,), jnp.float32),
            pltpu.SemaphoreType.DMA,
        ],
        compiler_params=pltpu.CompilerParams(use_tc_tiling_on_sc=False),
    )
    def retrieve(table_hbm, out_hbm, table_v, win_v, sem):
        c = lax.axis_index("c")
        s = lax.axis_index("s")
        wid = c * NS + s
        i0 = wid * rows_per_w

        pltpu.sync_copy(table_hbm, table_v)

        off = MAXREL - i0 - (rows_per_w - 1)

        def build_row(w, carry):
            src = jnp.clip(w + off, 0, T - 1) * D
            wd = w * D
            win_v[pl.ds(wd, 16)] = table_v[pl.ds(src, 16)]
            win_v[pl.ds(wd + 16, 16)] = table_v[pl.ds(src + 16, 16)]
            return carry

        lax.fori_loop(0, win, build_row, 0)

        copies = []
        for r in range(rows_per_w):
            copies.append(
                pltpu.async_copy(
                    win_v.at[pl.ds((rows_per_w - 1 - r) * D, L * D)],
                    out_hbm.at[pl.ds((i0 + r) * (L * D), L * D)],
                    sem,
                ))
        for cp in copies:
            cp.wait()

    return retrieve


def kernel(x, rel_embeds):
    L = x.shape[1]
    out_flat = _make_sc_kernel(L)(rel_embeds.reshape(-1))
    return out_flat.reshape(L, L, D)


# speedup vs baseline: 74.4120x; 9.1617x over previous
import functools

import jax
import jax.numpy as jnp
from jax import lax
from jax.experimental import pallas as pl
from jax.experimental.pallas import tpu as pltpu
from jax.experimental.pallas import tpu_sc as plsc

MAXREL = 128
D = 32
T = 2 * MAXREL + 1
NC = 2
NS = 16
NW = NC * NS
LANES = 16


def _make_sc_kernel(L: int):
    R = L // NW
    JT = L // 128
    NQ = 4
    BLK = L // NQ
    win_pad = L + BLK
    n_chunks = win_pad // LANES

    mesh = plsc.VectorSubcoreMesh(
        core_axis_name="c", subcore_axis_name="s",
        num_cores=NC, num_subcores=NS)

    @functools.partial(
        pl.kernel,
        out_type=jax.ShapeDtypeStruct((L, D // 8, JT, 8, 128), jnp.float32),
        mesh=mesh,
        scratch_types=[
            pltpu.VMEM((T * D,), jnp.float32),
            pltpu.VMEM((D // 8, 8, win_pad), jnp.float32),
            pltpu.SemaphoreType.DMA,
        ],
        compiler_params=pltpu.CompilerParams(
            use_tc_tiling_on_sc=False, needs_layout_passes=False),
    )
    def retrieve(table_hbm, out_hbm, table_v, sbt, sem):
        c = lax.axis_index("c")
        s = lax.axis_index("s")
        wid = c * NS + s
        p = wid & 7
        q = wid >> 3
        ibase = q * BLK + p

        pltpu.sync_copy(table_hbm, table_v)

        off = MAXREL - ibase - 8 * (R - 1)
        iota = lax.iota(jnp.int32, LANES)

        for d in range(D):
            dt, ds = d >> 3, d & 7

            def build_chunk(k, carry, d=d, dt=dt, ds=ds):
                c0 = k * LANES
                src = jnp.clip(c0 + iota + off, 0, T - 1)
                vals = plsc.load_gather(table_v, [src * D + d])
                sbt[dt, ds, pl.ds(c0, LANES)] = vals
                return carry

            lax.fori_loop(0, n_chunks, build_chunk, 0)

        def issue_row(t, carry):
            i = ibase + 8 * t
            cb = 8 * (R - 1 - t)
            for jt in range(JT):
                pltpu.async_copy(
                    sbt.at[:, :, pl.ds(cb + 128 * jt, 128)],
                    out_hbm.at[i, :, jt],
                    sem)
            return carry

        lax.fori_loop(0, R, issue_row, 0)

        def drain_row(t, carry):
            i = ibase + 8 * t
            cb = 8 * (R - 1 - t)
            for jt in range(JT):
                pltpu.make_async_copy(
                    sbt.at[:, :, pl.ds(cb + 128 * jt, 128)],
                    out_hbm.at[i, :, jt],
                    sem).wait()
            return carry

        lax.fori_loop(0, R, drain_row, 0)

    return retrieve


def kernel(x, rel_embeds):
    L = x.shape[1]
    out5 = _make_sc_kernel(L)(rel_embeds.reshape(-1))
    return out5.transpose(0, 2, 4, 1, 3).reshape(L, L, D)
